# baseline (speedup 1.0000x reference)
import numpy as np
import jax
import jax.numpy as jnp
from jax.experimental import pallas as pl
from jax.experimental.pallas import tpu as pltpu

_VOCAB = 100000
_SUB = 8
_LANES = _VOCAB // _SUB
_TINY = np.float32(np.finfo(np.float32).tiny)
_SPAN = np.float32(np.float32(1.0) - _TINY)

_ROT_A = (13, 15, 26, 6)
_ROT_B = (17, 29, 16, 24)


def _np_threefry2x32(k0, k1, x0, x1):
    m = 0xFFFFFFFF
    ks2 = (k0 ^ k1 ^ 0x1BD11BDA) & m
    x0 = (x0 + k0) & m
    x1 = (x1 + k1) & m
    sched = ((_ROT_A, k1, ks2, 1), (_ROT_B, ks2, k0, 2), (_ROT_A, k0, k1, 3),
             (_ROT_B, k1, ks2, 4), (_ROT_A, ks2, k0, 5))
    for rots, a0, a1, inc in sched:
        for r in rots:
            x0 = (x0 + x1) & m
            x1 = ((x1 << r) | (x1 >> (32 - r))) & m
            x1 = x0 ^ x1
        x0 = (x0 + a0) & m
        x1 = (x1 + a1 + inc) & m
    return x0, x1


_K1 = _np_threefry2x32(0, 42, 0, 1)
_K2 = _np_threefry2x32(0, 42, 0, 2)


def _rotl(x, r):
    return (x << jnp.uint32(r)) | (x >> jnp.uint32(32 - r))


def _threefry2x32(k0, k1, x0, x1):
    ks2 = k0 ^ k1 ^ jnp.uint32(0x1BD11BDA)
    x0 = x0 + k0
    x1 = x1 + k1
    sched = ((_ROT_A, k1, ks2, 1), (_ROT_B, ks2, k0, 2), (_ROT_A, k0, k1, 3),
             (_ROT_B, k1, ks2, 4), (_ROT_A, ks2, k0, 5))
    for rots, a0, a1, inc in sched:
        for r in rots:
            x0 = x0 + x1
            x1 = _rotl(x1, r) ^ x0
        x0 = x0 + a0
        x1 = x1 + a1 + jnp.uint32(inc)
    return x0, x1


def _bits(k0, k1, lo):
    hi = jnp.zeros_like(lo)
    o0, o1 = _threefry2x32(k0, k1, hi, lo)
    return o0 ^ o1


def _unit_float(bits):
    f = jax.lax.bitcast_convert_type(
        (bits >> jnp.uint32(9)) | jnp.uint32(0x3F800000), jnp.float32)
    return f - jnp.float32(1.0)


def _gather_dma(src_ref, dst_ref, i, t, sem):
    c = t % _LANES
    off = jnp.minimum((c // 128) * 128, _LANES - 128)
    off = pl.multiple_of(off, 128)
    return pltpu.make_async_copy(
        src_ref.at[pl.ds(i, 1), :, pl.ds(off, 128)],
        dst_ref.at[pl.ds(i, 1), :, :], sem)


def _body(dt_ref, dt17_ref, dp_ref, op_ref, tok_ref, na_ref,
          eb_d_ref, eb_o_ref, rowd_ref, rowo_ref, rowl_ref, sem_elm, sem_row):
    pltpu.make_async_copy(op_ref.at[pl.ds(16, 1)], rowl_ref, sem_row.at[2]).start()
    for i in range(16):
        t = dt_ref[i]
        _gather_dma(dp_ref, eb_d_ref, i, t, sem_elm.at[i]).start()
        _gather_dma(op_ref, eb_o_ref, i, t, sem_elm.at[16 + i]).start()
    for i in range(16):
        t = dt_ref[i]
        _gather_dma(dp_ref, eb_d_ref, i, t, sem_elm.at[i]).wait()
        _gather_dma(op_ref, eb_o_ref, i, t, sem_elm.at[16 + i]).wait()

    lo16 = jax.lax.broadcasted_iota(jnp.int32, (16, 1), 0).astype(jnp.uint32)
    uvec = _unit_float(_bits(jnp.uint32(0), jnp.uint32(42), lo16))
    idx16 = jax.lax.broadcasted_iota(jnp.int32, (16, 1), 0)
    sub_b = jax.lax.broadcasted_iota(jnp.int32, (1, _SUB, 128), 1)
    lane_b = jax.lax.broadcasted_iota(jnp.int32, (1, _SUB, 128), 2)
    first = jnp.int32(16)
    for i in range(16):
        t = dt_ref[i]
        r = t // _LANES
        c = t % _LANES
        off = jnp.minimum((c // 128) * 128, _LANES - 128)
        msk = (sub_b == r) & (lane_b == c - off)
        pd = jnp.sum(jnp.where(msk, eb_d_ref[pl.ds(i, 1)], 0.0))
        po = jnp.sum(jnp.where(msk, eb_o_ref[pl.ds(i, 1)], 0.0))
        u_i = jnp.sum(jnp.where(idx16 == i, uvec, 0.0))
        rej = u_i >= jnp.minimum(jnp.float32(1.0), po / pd)
        first = jnp.minimum(first, jnp.where(rej, jnp.int32(i), jnp.int32(16)))
    na = first
    j = jnp.minimum(first, jnp.int32(15))
    any_rej = first < jnp.int32(16)

    pltpu.make_async_copy(dp_ref.at[pl.ds(j, 1)], rowd_ref, sem_row.at[0]).start()
    pltpu.make_async_copy(op_ref.at[pl.ds(j, 1)], rowo_ref, sem_row.at[1]).start()

    k0 = jnp.where(any_rej, jnp.uint32(_K1[0]), jnp.uint32(_K2[0]))
    k1 = jnp.where(any_rej, jnp.uint32(_K1[1]), jnp.uint32(_K2[1]))
    sub = jax.lax.broadcasted_iota(jnp.int32, (_SUB, _LANES), 0)
    lane = jax.lax.broadcasted_iota(jnp.int32, (_SUB, _LANES), 1)
    e8 = sub * _LANES + lane
    bits = _bits(k0, k1, e8.astype(jnp.uint32))
    uu = jnp.maximum(_TINY, _unit_float(bits) * _SPAN + _TINY)
    g = -jnp.log(-jnp.log(uu))

    pltpu.make_async_copy(dp_ref.at[pl.ds(j, 1)], rowd_ref, sem_row.at[0]).wait()
    pltpu.make_async_copy(op_ref.at[pl.ds(j, 1)], rowo_ref, sem_row.at[1]).wait()
    pltpu.make_async_copy(op_ref.at[pl.ds(16, 1)], rowl_ref, sem_row.at[2]).wait()

    resid = rowo_ref[0] - rowd_ref[0]
    s = jnp.sum(jnp.sum(resid, axis=1))
    sel = jnp.where(any_rej, resid / s, rowl_ref[0])
    logits = jnp.log(jnp.clip(sel, jnp.float32(1e-20)))
    val = logits + g
    m = jnp.max(val)
    last = jnp.min(jnp.where(val == m, e8, jnp.int32(2147483647)))

    pos = jax.lax.broadcasted_iota(jnp.int32, (1, 17), 1)
    base = jnp.where(pos < na, dt17_ref[...], jnp.int32(-1))
    tok_ref[...] = jnp.where(pos == na, last, base)
    na_ref[0, 0] = na


def kernel(draft_tokens, draft_probs, oracle_tokens, oracle_probs, num_draft_tokens):
    del oracle_tokens, num_draft_tokens
    dt = draft_tokens.astype(jnp.int32)
    dt17 = jnp.concatenate([dt, jnp.full((1,), -1, jnp.int32)]).reshape(1, 17)
    dp3 = draft_probs.reshape(16, _SUB, _LANES)
    op3 = oracle_probs.reshape(17, _SUB, _LANES)
    tok, na = pl.pallas_call(
        _body,
        out_shape=(
            jax.ShapeDtypeStruct((1, 17), jnp.int32),
            jax.ShapeDtypeStruct((1, 1), jnp.int32),
        ),
        in_specs=[
            pl.BlockSpec(memory_space=pltpu.SMEM),
            pl.BlockSpec(memory_space=pltpu.VMEM),
            pl.BlockSpec(memory_space=pltpu.MemorySpace.HBM),
            pl.BlockSpec(memory_space=pltpu.MemorySpace.HBM),
        ],
        out_specs=(
            pl.BlockSpec(memory_space=pltpu.VMEM),
            pl.BlockSpec(memory_space=pltpu.SMEM),
        ),
        scratch_shapes=[
            pltpu.VMEM((16, _SUB, 128), jnp.float32),
            pltpu.VMEM((16, _SUB, 128), jnp.float32),
            pltpu.VMEM((1, _SUB, _LANES), jnp.float32),
            pltpu.VMEM((1, _SUB, _LANES), jnp.float32),
            pltpu.VMEM((1, _SUB, _LANES), jnp.float32),
            pltpu.SemaphoreType.DMA((32,)),
            pltpu.SemaphoreType.DMA((3,)),
        ],
    )(dt, dt17, dp3, op3)
    return tok.reshape(17).astype(draft_tokens.dtype), na.reshape(())


# speedup vs baseline: 1.6865x; 1.6865x over previous
import numpy as np
import jax
import jax.numpy as jnp
from jax.experimental import pallas as pl
from jax.experimental.pallas import tpu as pltpu

_VOCAB = 100000
_SUB = 8
_LANES = _VOCAB // _SUB
_TAIL = 99968
_TINY = np.float32(np.finfo(np.float32).tiny)
_SPAN = np.float32(np.float32(1.0) - _TINY)

_ROT_A = (13, 15, 26, 6)
_ROT_B = (17, 29, 16, 24)


def _np_threefry2x32(k0, k1, x0, x1):
    m = 0xFFFFFFFF
    ks2 = (k0 ^ k1 ^ 0x1BD11BDA) & m
    x0 = (x0 + k0) & m
    x1 = (x1 + k1) & m
    sched = ((_ROT_A, k1, ks2, 1), (_ROT_B, ks2, k0, 2), (_ROT_A, k0, k1, 3),
             (_ROT_B, k1, ks2, 4), (_ROT_A, ks2, k0, 5))
    for rots, a0, a1, inc in sched:
        for r in rots:
            x0 = (x0 + x1) & m
            x1 = ((x1 << r) | (x1 >> (32 - r))) & m
            x1 = x0 ^ x1
        x0 = (x0 + a0) & m
        x1 = (x1 + a1 + inc) & m
    return x0, x1


_K1 = _np_threefry2x32(0, 42, 0, 1)
_K2 = _np_threefry2x32(0, 42, 0, 2)


def _rotl(x, r):
    return (x << jnp.uint32(r)) | (x >> jnp.uint32(32 - r))


def _threefry2x32(k0, k1, x0, x1):
    ks2 = k0 ^ k1 ^ jnp.uint32(0x1BD11BDA)
    x0 = x0 + k0
    x1 = x1 + k1
    sched = ((_ROT_A, k1, ks2, 1), (_ROT_B, ks2, k0, 2), (_ROT_A, k0, k1, 3),
             (_ROT_B, k1, ks2, 4), (_ROT_A, ks2, k0, 5))
    for rots, a0, a1, inc in sched:
        for r in rots:
            x0 = x0 + x1
            x1 = _rotl(x1, r) ^ x0
        x0 = x0 + a0
        x1 = x1 + a1 + jnp.uint32(inc)
    return x0, x1


def _bits(k0, k1, lo):
    hi = jnp.zeros_like(lo)
    o0, o1 = _threefry2x32(k0, k1, hi, lo)
    return o0 ^ o1


def _unit_float(bits):
    f = jax.lax.bitcast_convert_type(
        (bits >> jnp.uint32(9)) | jnp.uint32(0x3F800000), jnp.float32)
    return f - jnp.float32(1.0)


def _blk_dma(src_ref, dst_ref, i, t, sem):
    off = jnp.minimum((t // 128) * 128, jnp.int32(_TAIL - 128))
    off = pl.multiple_of(off, 128)
    i8 = (i // 8) * 8
    return pltpu.make_async_copy(
        src_ref.at[pl.ds(i8, 8), pl.ds(off, 128)],
        dst_ref.at[pl.ds(i * 8, 8), :], sem)


def _body(dt_ref, dt17_ref, dp_ref, op_ref, dtl_ref, otl_ref, opl_ref,
          tok_ref, na_ref,
          eb_d_ref, eb_o_ref, dpg_ref, opg_ref, res_ref, opp_ref,
          sem_elm, sem_row):
    for i in range(16):
        t = dt_ref[i]
        _blk_dma(dp_ref, eb_d_ref, i, t, sem_elm.at[i]).start()
        _blk_dma(op_ref, eb_o_ref, i, t, sem_elm.at[16 + i]).start()

    lo16 = jax.lax.broadcasted_iota(jnp.int32, (16, 1), 0).astype(jnp.uint32)
    uvec = _unit_float(_bits(jnp.uint32(0), jnp.uint32(42), lo16))
    idx16 = jax.lax.broadcasted_iota(jnp.int32, (16, 1), 0)
    sub_b = jax.lax.broadcasted_iota(jnp.int32, (8, 128), 0)
    lane_b = jax.lax.broadcasted_iota(jnp.int32, (8, 128), 1)
    lane_t = jax.lax.broadcasted_iota(jnp.int32, (16, _VOCAB - _TAIL), 1)

    for i in range(16):
        t = dt_ref[i]
        _blk_dma(dp_ref, eb_d_ref, i, t, sem_elm.at[i]).wait()
        _blk_dma(op_ref, eb_o_ref, i, t, sem_elm.at[16 + i]).wait()

    first = jnp.int32(16)
    for i in range(16):
        t = dt_ref[i]
        off = jnp.minimum((t // 128) * 128, jnp.int32(_TAIL - 128))
        msk = (sub_b == i % 8) & (lane_b == t - off)
        pd_b = jnp.sum(jnp.where(msk, eb_d_ref[pl.ds(i * 8, 8), :], 0.0))
        po_b = jnp.sum(jnp.where(msk, eb_o_ref[pl.ds(i * 8, 8), :], 0.0))
        mskt = (idx16 == i) & (lane_t == t - _TAIL)
        pd_t = jnp.sum(jnp.where(mskt, dtl_ref[...], 0.0))
        po_t = jnp.sum(jnp.where(mskt, otl_ref[...], 0.0))
        in_tail = t >= _TAIL
        pd = jnp.where(in_tail, pd_t, pd_b)
        po = jnp.where(in_tail, po_t, po_b)
        u_i = jnp.sum(jnp.where(idx16 == i, uvec, 0.0))
        rej = u_i >= jnp.minimum(jnp.float32(1.0), po / pd)
        first = jnp.minimum(first, jnp.where(rej, jnp.int32(i), jnp.int32(16)))
    na = first
    j = jnp.minimum(first, jnp.int32(15))
    any_rej = first < jnp.int32(16)

    jm = j % 8
    j8 = pl.multiple_of(j - jm, 8)
    pltpu.make_async_copy(dp_ref.at[pl.ds(j8, 8), :], dpg_ref, sem_row.at[0]).start()
    pltpu.make_async_copy(op_ref.at[pl.ds(j8, 8), :], opg_ref, sem_row.at[1]).start()

    k0 = jnp.where(any_rej, jnp.uint32(_K1[0]), jnp.uint32(_K2[0]))
    k1 = jnp.where(any_rej, jnp.uint32(_K1[1]), jnp.uint32(_K2[1]))
    sub = jax.lax.broadcasted_iota(jnp.int32, (_SUB, _LANES), 0)
    lane = jax.lax.broadcasted_iota(jnp.int32, (_SUB, _LANES), 1)
    e8 = sub * _LANES + lane
    bits = _bits(k0, k1, e8.astype(jnp.uint32))
    uu = jnp.maximum(_TINY, _unit_float(bits) * _SPAN + _TINY)
    g = -jnp.log(-jnp.log(uu))

    pltpu.make_async_copy(dp_ref.at[pl.ds(j8, 8), :], dpg_ref, sem_row.at[0]).wait()
    pltpu.make_async_copy(op_ref.at[pl.ds(j8, 8), :], opg_ref, sem_row.at[1]).wait()

    for r in range(_SUB):
        ch = pl.ds(r * _LANES, _LANES)
        res_ref[pl.ds(r, 1), :] = (opg_ref[pl.ds(jm, 1), ch]
                                   - dpg_ref[pl.ds(jm, 1), ch])
        opp_ref[pl.ds(r, 1), :] = opl_ref[0:1, ch]

    resid = res_ref[...]
    s = jnp.sum(jnp.sum(resid, axis=1))
    sel = jnp.where(any_rej, resid / s, opp_ref[...])
    logits = jnp.log(jnp.clip(sel, jnp.float32(1e-20)))
    val = logits + g
    m = jnp.max(val)
    last = jnp.min(jnp.where(val == m, e8, jnp.int32(2147483647)))

    pos = jax.lax.broadcasted_iota(jnp.int32, (1, 17), 1)
    base = jnp.where(pos < na, dt17_ref[...], jnp.int32(-1))
    tok_ref[...] = jnp.where(pos == na, last, base)
    na_ref[0, 0] = na


def kernel(draft_tokens, draft_probs, oracle_tokens, oracle_probs, num_draft_tokens):
    del oracle_tokens, num_draft_tokens
    dt = draft_tokens.astype(jnp.int32)
    dt17 = jnp.concatenate([dt, jnp.full((1,), -1, jnp.int32)]).reshape(1, 17)
    dp_tail = draft_probs[:, _TAIL:]
    op_tail = oracle_probs[:16, _TAIL:]
    op_last = oracle_probs[16:17, :]
    tok, na = pl.pallas_call(
        _body,
        out_shape=(
            jax.ShapeDtypeStruct((1, 17), jnp.int32),
            jax.ShapeDtypeStruct((1, 1), jnp.int32),
        ),
        in_specs=[
            pl.BlockSpec(memory_space=pltpu.SMEM),
            pl.BlockSpec(memory_space=pltpu.VMEM),
            pl.BlockSpec(memory_space=pltpu.MemorySpace.HBM),
            pl.BlockSpec(memory_space=pltpu.MemorySpace.HBM),
            pl.BlockSpec(memory_space=pltpu.VMEM),
            pl.BlockSpec(memory_space=pltpu.VMEM),
            pl.BlockSpec(memory_space=pltpu.VMEM),
        ],
        out_specs=(
            pl.BlockSpec(memory_space=pltpu.VMEM),
            pl.BlockSpec(memory_space=pltpu.SMEM),
        ),
        scratch_shapes=[
            pltpu.VMEM((128, 128), jnp.float32),
            pltpu.VMEM((128, 128), jnp.float32),
            pltpu.VMEM((_SUB, _VOCAB), jnp.float32),
            pltpu.VMEM((_SUB, _VOCAB), jnp.float32),
            pltpu.VMEM((_SUB, _LANES), jnp.float32),
            pltpu.VMEM((_SUB, _LANES), jnp.float32),
            pltpu.SemaphoreType.DMA((32,)),
            pltpu.SemaphoreType.DMA((2,)),
        ],
    )(dt, dt17, draft_probs, oracle_probs, dp_tail, op_tail, op_last)
    return tok.reshape(17).astype(draft_tokens.dtype), na.reshape(())


# speedup vs baseline: 1.7246x; 1.0226x over previous
import numpy as np
import jax
import jax.numpy as jnp
from jax.experimental import pallas as pl
from jax.experimental.pallas import tpu as pltpu

_VOCAB = 100000
_SUB = 8
_LANES = _VOCAB // _SUB
_TAIL = 99968
_TINY = np.float32(np.finfo(np.float32).tiny)
_SPAN = np.float32(np.float32(1.0) - _TINY)

_ROT_A = (13, 15, 26, 6)
_ROT_B = (17, 29, 16, 24)


def _np_threefry2x32(k0, k1, x0, x1):
    m = 0xFFFFFFFF
    ks2 = (k0 ^ k1 ^ 0x1BD11BDA) & m
    x0 = (x0 + k0) & m
    x1 = (x1 + k1) & m
    sched = ((_ROT_A, k1, ks2, 1), (_ROT_B, ks2, k0, 2), (_ROT_A, k0, k1, 3),
             (_ROT_B, k1, ks2, 4), (_ROT_A, ks2, k0, 5))
    for rots, a0, a1, inc in sched:
        for r in rots:
            x0 = (x0 + x1) & m
            x1 = ((x1 << r) | (x1 >> (32 - r))) & m
            x1 = x0 ^ x1
        x0 = (x0 + a0) & m
        x1 = (x1 + a1 + inc) & m
    return x0, x1


_K1 = _np_threefry2x32(0, 42, 0, 1)
_K2 = _np_threefry2x32(0, 42, 0, 2)


def _rotl(x, r):
    return (x << jnp.uint32(r)) | (x >> jnp.uint32(32 - r))


def _threefry2x32(k0, k1, x0, x1):
    ks2 = k0 ^ k1 ^ jnp.uint32(0x1BD11BDA)
    x0 = x0 + k0
    x1 = x1 + k1
    sched = ((_ROT_A, k1, ks2, 1), (_ROT_B, ks2, k0, 2), (_ROT_A, k0, k1, 3),
             (_ROT_B, k1, ks2, 4), (_ROT_A, ks2, k0, 5))
    for rots, a0, a1, inc in sched:
        for r in rots:
            x0 = x0 + x1
            x1 = _rotl(x1, r) ^ x0
        x0 = x0 + a0
        x1 = x1 + a1 + jnp.uint32(inc)
    return x0, x1


def _bits(k0, k1, lo):
    hi = jnp.zeros_like(lo)
    o0, o1 = _threefry2x32(k0, k1, hi, lo)
    return o0 ^ o1


def _unit_float(bits):
    f = jax.lax.bitcast_convert_type(
        (bits >> jnp.uint32(9)) | jnp.uint32(0x3F800000), jnp.float32)
    return f - jnp.float32(1.0)


def _blk_dma(src_ref, dst_ref, i, t, sem):
    off = jnp.minimum((t // 128) * 128, jnp.int32(_TAIL - 128))
    off = pl.multiple_of(off, 128)
    i8 = (i // 8) * 8
    return pltpu.make_async_copy(
        src_ref.at[pl.ds(i8, 8), pl.ds(off, 128)],
        dst_ref.at[pl.ds(i * 8, 8), :], sem)


def _body(dt_ref, dt17_ref, dp_ref, op_ref, dtl_ref, otl_ref, opl_ref,
          tok_ref, na_ref,
          eb_d_ref, eb_o_ref, dpg_ref, opg_ref, res_ref, opl_v_ref,
          sem_elm, sem_row):
    for i in range(16):
        t = dt_ref[i]
        _blk_dma(dp_ref, eb_d_ref, i, t, sem_elm.at[i]).start()
        _blk_dma(op_ref, eb_o_ref, i, t, sem_elm.at[16 + i]).start()

    lo16 = jax.lax.broadcasted_iota(jnp.int32, (16, 1), 0).astype(jnp.uint32)
    uvec = _unit_float(_bits(jnp.uint32(0), jnp.uint32(42), lo16))
    idx16 = jax.lax.broadcasted_iota(jnp.int32, (16, 1), 0)
    sub_b = jax.lax.broadcasted_iota(jnp.int32, (8, 128), 0)
    lane_b = jax.lax.broadcasted_iota(jnp.int32, (8, 128), 1)
    lane_t = jax.lax.broadcasted_iota(jnp.int32, (16, _VOCAB - _TAIL), 1)

    for i in range(16):
        t = dt_ref[i]
        _blk_dma(dp_ref, eb_d_ref, i, t, sem_elm.at[i]).wait()
        _blk_dma(op_ref, eb_o_ref, i, t, sem_elm.at[16 + i]).wait()

    first = jnp.int32(16)
    for i in range(16):
        t = dt_ref[i]
        off = jnp.minimum((t // 128) * 128, jnp.int32(_TAIL - 128))
        msk = (sub_b == i % 8) & (lane_b == t - off)
        pd_b = jnp.sum(jnp.where(msk, eb_d_ref[pl.ds(i * 8, 8), :], 0.0))
        po_b = jnp.sum(jnp.where(msk, eb_o_ref[pl.ds(i * 8, 8), :], 0.0))
        mskt = (idx16 == i) & (lane_t == t - _TAIL)
        pd_t = jnp.sum(jnp.where(mskt, dtl_ref[...], 0.0))
        po_t = jnp.sum(jnp.where(mskt, otl_ref[...], 0.0))
        in_tail = t >= _TAIL
        pd = jnp.where(in_tail, pd_t, pd_b)
        po = jnp.where(in_tail, po_t, po_b)
        u_i = jnp.sum(jnp.where(idx16 == i, uvec, 0.0))
        rej = u_i >= jnp.minimum(jnp.float32(1.0), po / pd)
        first = jnp.minimum(first, jnp.where(rej, jnp.int32(i), jnp.int32(16)))
    na = first
    j = jnp.minimum(first, jnp.int32(15))
    any_rej = first < jnp.int32(16)

    jm = j % 8
    j8 = pl.multiple_of(j - jm, 8)
    pltpu.make_async_copy(dp_ref.at[pl.ds(j8, 8), :], dpg_ref, sem_row.at[0]).start()
    pltpu.make_async_copy(op_ref.at[pl.ds(j8, 8), :], opg_ref, sem_row.at[1]).start()

    k0 = jnp.where(any_rej, jnp.uint32(_K1[0]), jnp.uint32(_K2[0]))
    k1 = jnp.where(any_rej, jnp.uint32(_K1[1]), jnp.uint32(_K2[1]))
    sub = jax.lax.broadcasted_iota(jnp.int32, (_SUB, _LANES), 0)
    lane = jax.lax.broadcasted_iota(jnp.int32, (_SUB, _LANES), 1)
    e8 = sub * _LANES + lane
    bits = _bits(k0, k1, e8.astype(jnp.uint32))
    uu = jnp.maximum(_TINY, _unit_float(bits) * _SPAN + _TINY)
    g = -jnp.log(-jnp.log(uu))

    pltpu.make_async_copy(dp_ref.at[pl.ds(j8, 8), :], dpg_ref, sem_row.at[0]).wait()
    pltpu.make_async_copy(op_ref.at[pl.ds(j8, 8), :], opg_ref, sem_row.at[1]).wait()

    @pl.when(any_rej)
    def _resid_path():
        for r in range(_SUB):
            ch = pl.ds(r * _LANES, _LANES)
            res_ref[pl.ds(r, 1), :] = (opg_ref[pl.ds(jm, 1), ch]
                                       - dpg_ref[pl.ds(jm, 1), ch])
        resid = res_ref[...]
        s = jnp.sum(jnp.sum(resid, axis=1))
        res_ref[...] = resid / s

    @pl.when(jnp.logical_not(any_rej))
    def _bonus_path():
        pltpu.make_async_copy(opl_ref, opl_v_ref, sem_row.at[2]).start()
        pltpu.make_async_copy(opl_ref, opl_v_ref, sem_row.at[2]).wait()
        for r in range(_SUB):
            res_ref[pl.ds(r, 1), :] = opl_v_ref[0:1, pl.ds(r * _LANES, _LANES)]

    logits = jnp.log(jnp.clip(res_ref[...], jnp.float32(1e-20)))
    val = logits + g
    m = jnp.max(val)
    last = jnp.min(jnp.where(val == m, e8, jnp.int32(2147483647)))

    pos = jax.lax.broadcasted_iota(jnp.int32, (1, 17), 1)
    base = jnp.where(pos < na, dt17_ref[...], jnp.int32(-1))
    tok_ref[...] = jnp.where(pos == na, last, base)
    na_ref[0, 0] = na


def kernel(draft_tokens, draft_probs, oracle_tokens, oracle_probs, num_draft_tokens):
    del oracle_tokens, num_draft_tokens
    dt = draft_tokens.astype(jnp.int32)
    dt17 = jnp.concatenate([dt, jnp.full((1,), -1, jnp.int32)]).reshape(1, 17)
    dp_tail = draft_probs[:, _TAIL:]
    op_tail = oracle_probs[:16, _TAIL:]
    op_last = oracle_probs[16:17, :]
    tok, na = pl.pallas_call(
        _body,
        out_shape=(
            jax.ShapeDtypeStruct((1, 17), jnp.int32),
            jax.ShapeDtypeStruct((1, 1), jnp.int32),
        ),
        in_specs=[
            pl.BlockSpec(memory_space=pltpu.SMEM),
            pl.BlockSpec(memory_space=pltpu.VMEM),
            pl.BlockSpec(memory_space=pltpu.MemorySpace.HBM),
            pl.BlockSpec(memory_space=pltpu.MemorySpace.HBM),
            pl.BlockSpec(memory_space=pltpu.VMEM),
            pl.BlockSpec(memory_space=pltpu.VMEM),
            pl.BlockSpec(memory_space=pltpu.MemorySpace.HBM),
        ],
        out_specs=(
            pl.BlockSpec(memory_space=pltpu.VMEM),
            pl.BlockSpec(memory_space=pltpu.SMEM),
        ),
        scratch_shapes=[
            pltpu.VMEM((128, 128), jnp.float32),
            pltpu.VMEM((128, 128), jnp.float32),
            pltpu.VMEM((_SUB, _VOCAB), jnp.float32),
            pltpu.VMEM((_SUB, _VOCAB), jnp.float32),
            pltpu.VMEM((_SUB, _LANES), jnp.float32),
            pltpu.VMEM((1, _VOCAB), jnp.float32),
            pltpu.SemaphoreType.DMA((32,)),
            pltpu.SemaphoreType.DMA((3,)),
        ],
    )(dt, dt17, draft_probs, oracle_probs, dp_tail, op_tail, op_last)
    return tok.reshape(17).astype(draft_tokens.dtype), na.reshape(())
